# baseline (speedup 1.0000x reference)
import jax
import jax.numpy as jnp
from jax.experimental import pallas as pl

_EDGE_SRC = (0, 1, 2, 3, 0, 2, 1, 4, 3, 4)
_EDGE_DST = (1, 0, 3, 2, 2, 0, 4, 1, 4, 3)
_N = 5
_E = 10
_H = 4
_C = 64
_TB = 512


def _gat_kernel(s_id_ref, s_st_ref, b_id_ref, b_st_ref, part_ref, hs_ref,
                WcT_ref, bc_ref, WlT_ref, bl_ref, WrT_ref, br_ref,
                att_ref, gbias_ref, lnw_ref, lnb_ref,
                out_ref, attn_ref):
    f32 = jnp.float32
    bf16 = jnp.bfloat16
    hs = hs_ref[...]
    tb = hs.shape[0]
    ctx = jnp.dot(hs.astype(bf16), WcT_ref[...],
                  preferred_element_type=f32) + bc_ref[...]

    WlT = WlT_ref[...]
    WrT = WrT_ref[...]
    bl = bl_ref[...]
    br = br_ref[...]
    xl = []
    xr = []
    for ref in (s_id_ref, s_st_ref, b_id_ref, b_st_ref, part_ref):
        n = (ref[...] + ctx).astype(bf16)
        xl.append(jnp.dot(n, WlT, preferred_element_type=f32) + bl)
        xr.append(jnp.dot(n, WrT, preferred_element_type=f32) + br)

    att = att_ref[...]
    alphas = []
    for e in range(_E):
        pre = xl[_EDGE_SRC[e]] + xr[_EDGE_DST[e]]
        act = jnp.where(pre >= 0, pre, 0.2 * pre)
        prod = act * att
        alphas.append(jnp.sum(prod.reshape(tb, _H, _C), axis=2))

    in_edges = [[] for _ in range(_N)]
    for e, d in enumerate(_EDGE_DST):
        in_edges[d].append(e)
    a = [None] * _E
    for d in range(_N):
        e1, e2 = in_edges[d]
        m = jnp.maximum(alphas[e1], alphas[e2])
        x1 = jnp.exp(alphas[e1] - m)
        x2 = jnp.exp(alphas[e2] - m)
        inv = 1.0 / (x1 + x2 + 1e-16)
        a[e1] = x1 * inv
        a[e2] = x2 * inv

    acc = None
    for i in range(_N):
        es = [e for e in range(_E) if _EDGE_SRC[e] == i]
        w = a[es[0]] + a[es[1]]
        wb = jnp.broadcast_to(w[:, :, None], (tb, _H, _C)).reshape(tb, _H * _C)
        term = xl[i] * wb
        acc = term if acc is None else acc + term
    h = acc * 0.2 + gbias_ref[...]

    mu = jnp.mean(h, axis=1, keepdims=True)
    cen = h - mu
    var = jnp.mean(cen * cen, axis=1, keepdims=True)
    out_ref[...] = cen * jax.lax.rsqrt(var + 1e-5) * lnw_ref[...] + lnb_ref[...]

    attn_ref[0] = jnp.concatenate([a[e][0:1, :] for e in range(_E)], axis=1)


def kernel(striker_identity, striker_state, bowler_identity, bowler_state,
           partnership, h_state, Wc, bc, Wl, bl, Wr, br, att, gat_bias,
           ln_w, ln_b):
    B, D = h_state.shape
    HC = Wl.shape[0]
    grid = B // _TB
    row = lambda v: v.reshape(1, -1)

    h_actor, attn_all = pl.pallas_call(
        _gat_kernel,
        grid=(grid,),
        in_specs=[
            pl.BlockSpec((_TB, D), lambda i: (i, 0)),
            pl.BlockSpec((_TB, D), lambda i: (i, 0)),
            pl.BlockSpec((_TB, D), lambda i: (i, 0)),
            pl.BlockSpec((_TB, D), lambda i: (i, 0)),
            pl.BlockSpec((_TB, D), lambda i: (i, 0)),
            pl.BlockSpec((_TB, D), lambda i: (i, 0)),
            pl.BlockSpec((D, D), lambda i: (0, 0)),
            pl.BlockSpec((1, D), lambda i: (0, 0)),
            pl.BlockSpec((D, HC), lambda i: (0, 0)),
            pl.BlockSpec((1, HC), lambda i: (0, 0)),
            pl.BlockSpec((D, HC), lambda i: (0, 0)),
            pl.BlockSpec((1, HC), lambda i: (0, 0)),
            pl.BlockSpec((1, HC), lambda i: (0, 0)),
            pl.BlockSpec((1, HC), lambda i: (0, 0)),
            pl.BlockSpec((1, D), lambda i: (0, 0)),
            pl.BlockSpec((1, D), lambda i: (0, 0)),
        ],
        out_specs=[
            pl.BlockSpec((_TB, D), lambda i: (i, 0)),
            pl.BlockSpec((1, 1, _E * _H), lambda i: (i, 0, 0)),
        ],
        out_shape=[
            jax.ShapeDtypeStruct((B, D), jnp.float32),
            jax.ShapeDtypeStruct((grid, 1, _E * _H), jnp.float32),
        ],
    )(striker_identity, striker_state, bowler_identity, bowler_state,
      partnership, h_state, Wc.T.astype(jnp.bfloat16), row(bc),
      Wl.T.astype(jnp.bfloat16), row(bl), Wr.T.astype(jnp.bfloat16), row(br),
      att.reshape(1, -1), row(gat_bias), row(ln_w), row(ln_b))

    attn0 = attn_all[0, 0].reshape(_E, _H)
    return h_actor, attn0


# speedup vs baseline: 2.1598x; 2.1598x over previous
import jax
import jax.numpy as jnp
from jax.experimental import pallas as pl

_EDGE_SRC = (0, 1, 2, 3, 0, 2, 1, 4, 3, 4)
_EDGE_DST = (1, 0, 3, 2, 2, 0, 4, 1, 4, 3)
_N = 5
_E = 10
_H = 4
_C = 64
_TB = 512


def _gat_kernel(s_id_ref, s_st_ref, b_id_ref, b_st_ref, part_ref, hs_ref,
                WcT_ref, bc_ref, WlT_ref, bl_ref, WrT_ref, br_ref,
                att_ref, gbias_ref, lnw_ref, lnb_ref,
                out_ref, attn_ref):
    f32 = jnp.float32
    bf16 = jnp.bfloat16
    hs = hs_ref[...]
    tb = hs.shape[0]
    ctx = jnp.dot(hs.astype(bf16), WcT_ref[...],
                  preferred_element_type=f32) + bc_ref[...]

    WlT = WlT_ref[...]
    WrT = WrT_ref[...]
    bl = bl_ref[...]
    br = br_ref[...]
    xl = []
    xr = []
    for ref in (s_id_ref, s_st_ref, b_id_ref, b_st_ref, part_ref):
        n = (ref[...] + ctx).astype(bf16)
        xl.append(jnp.dot(n, WlT, preferred_element_type=f32) + bl)
        xr.append(jnp.dot(n, WrT, preferred_element_type=f32) + br)

    A_att = att_ref[...]
    xl_bf = [x.astype(bf16) for x in xl]
    xr_bf = [x.astype(bf16) for x in xr]
    alphas = []
    for e in range(_E):
        pre = xl_bf[_EDGE_SRC[e]] + xr_bf[_EDGE_DST[e]]
        act = jnp.maximum(pre, 0.2 * pre)
        alphas.append(jnp.dot(act, A_att, preferred_element_type=f32))

    in_edges = [[] for _ in range(_N)]
    for e, d in enumerate(_EDGE_DST):
        in_edges[d].append(e)
    a = [None] * _E
    for d in range(_N):
        e1, e2 = in_edges[d]
        m = jnp.maximum(alphas[e1], alphas[e2])
        x1 = jnp.exp(alphas[e1] - m)
        x2 = jnp.exp(alphas[e2] - m)
        inv = 1.0 / (x1 + x2 + 1e-16)
        a[e1] = x1 * inv
        a[e2] = x2 * inv

    acc = None
    for i in range(_N):
        es = [e for e in range(_E) if _EDGE_SRC[e] == i]
        w = a[es[0]] + a[es[1]]
        wb = jnp.broadcast_to(w[:, :, None], (tb, _H, _C)).reshape(tb, _H * _C)
        term = xl[i] * wb
        acc = term if acc is None else acc + term
    h = acc * 0.2 + gbias_ref[...]

    mu = jnp.mean(h, axis=1, keepdims=True)
    cen = h - mu
    var = jnp.mean(cen * cen, axis=1, keepdims=True)
    out_ref[...] = cen * jax.lax.rsqrt(var + 1e-5) * lnw_ref[...] + lnb_ref[...]

    attn_ref[0] = jnp.concatenate([a[e][0:1, :] for e in range(_E)], axis=1)


def _att_selector(att):
    eye = jnp.eye(_H, dtype=att.dtype)
    sel = eye[:, None, :] * att[:, :, None]
    return sel.reshape(_H * _C, _H).astype(jnp.bfloat16)


def kernel(striker_identity, striker_state, bowler_identity, bowler_state,
           partnership, h_state, Wc, bc, Wl, bl, Wr, br, att, gat_bias,
           ln_w, ln_b):
    B, D = h_state.shape
    HC = Wl.shape[0]
    grid = B // _TB
    row = lambda v: v.reshape(1, -1)

    h_actor, attn_all = pl.pallas_call(
        _gat_kernel,
        grid=(grid,),
        in_specs=[
            pl.BlockSpec((_TB, D), lambda i: (i, 0)),
            pl.BlockSpec((_TB, D), lambda i: (i, 0)),
            pl.BlockSpec((_TB, D), lambda i: (i, 0)),
            pl.BlockSpec((_TB, D), lambda i: (i, 0)),
            pl.BlockSpec((_TB, D), lambda i: (i, 0)),
            pl.BlockSpec((_TB, D), lambda i: (i, 0)),
            pl.BlockSpec((D, D), lambda i: (0, 0)),
            pl.BlockSpec((1, D), lambda i: (0, 0)),
            pl.BlockSpec((D, HC), lambda i: (0, 0)),
            pl.BlockSpec((1, HC), lambda i: (0, 0)),
            pl.BlockSpec((D, HC), lambda i: (0, 0)),
            pl.BlockSpec((1, HC), lambda i: (0, 0)),
            pl.BlockSpec((HC, _H), lambda i: (0, 0)),
            pl.BlockSpec((1, HC), lambda i: (0, 0)),
            pl.BlockSpec((1, D), lambda i: (0, 0)),
            pl.BlockSpec((1, D), lambda i: (0, 0)),
        ],
        out_specs=[
            pl.BlockSpec((_TB, D), lambda i: (i, 0)),
            pl.BlockSpec((1, 1, _E * _H), lambda i: (i, 0, 0)),
        ],
        out_shape=[
            jax.ShapeDtypeStruct((B, D), jnp.float32),
            jax.ShapeDtypeStruct((grid, 1, _E * _H), jnp.float32),
        ],
    )(striker_identity, striker_state, bowler_identity, bowler_state,
      partnership, h_state, Wc.T.astype(jnp.bfloat16), row(bc),
      Wl.T.astype(jnp.bfloat16), row(bl), Wr.T.astype(jnp.bfloat16), row(br),
      _att_selector(att), row(gat_bias), row(ln_w), row(ln_b))

    attn0 = attn_all[0, 0].reshape(_E, _H)
    return h_actor, attn0


# speedup vs baseline: 4.2081x; 1.9484x over previous
import jax
import jax.numpy as jnp
from jax.experimental import pallas as pl

_EDGE_SRC = (0, 1, 2, 3, 0, 2, 1, 4, 3, 4)
_EDGE_DST = (1, 0, 3, 2, 2, 0, 4, 1, 4, 3)
_N = 5
_E = 10
_H = 4
_C = 64
_TB = 512


def _gat_kernel(s_id_ref, s_st_ref, b_id_ref, b_st_ref, part_ref, hs_ref,
                WcT_ref, bc_ref, WlT_ref, bl_ref, WrT_ref, br_ref,
                att_ref, bcast_ref, ones_ref, gbias_ref, lnw_ref, lnb_ref,
                out_ref, attn_ref):
    f32 = jnp.float32
    bf16 = jnp.bfloat16
    hs = hs_ref[...]
    tb = hs.shape[0]
    ctx = jnp.dot(hs.astype(bf16), WcT_ref[...],
                  preferred_element_type=f32) + bc_ref[...]

    WlT = WlT_ref[...]
    WrT = WrT_ref[...]
    bl = bl_ref[...]
    br = br_ref[...]
    xl = []
    xr = []
    for ref in (s_id_ref, s_st_ref, b_id_ref, b_st_ref, part_ref):
        n = (ref[...] + ctx).astype(bf16)
        xl.append(jnp.dot(n, WlT, preferred_element_type=f32) + bl)
        xr.append(jnp.dot(n, WrT, preferred_element_type=f32) + br)

    A_att = att_ref[...]
    xl_bf = [x.astype(bf16) for x in xl]
    xr_bf = [x.astype(bf16) for x in xr]
    alphas = []
    for e in range(_E):
        pre = xl_bf[_EDGE_SRC[e]] + xr_bf[_EDGE_DST[e]]
        act = jnp.maximum(pre, 0.2 * pre)
        alphas.append(jnp.dot(act, A_att, preferred_element_type=f32))

    in_edges = [[] for _ in range(_N)]
    for e, d in enumerate(_EDGE_DST):
        in_edges[d].append(e)
    a = [None] * _E
    for d in range(_N):
        e1, e2 = in_edges[d]
        m = jnp.maximum(alphas[e1], alphas[e2])
        x1 = jnp.exp(alphas[e1] - m)
        x2 = jnp.exp(alphas[e2] - m)
        inv = 1.0 / (x1 + x2 + 1e-16)
        a[e1] = x1 * inv
        a[e2] = x2 * inv

    bcast = bcast_ref[...]
    acc = None
    for i in range(_N):
        es = [e for e in range(_E) if _EDGE_SRC[e] == i]
        w = a[es[0]] + a[es[1]]
        wb = jnp.dot(w.astype(bf16), bcast, preferred_element_type=f32)
        term = xl[i] * wb
        acc = term if acc is None else acc + term
    h = acc * 0.2 + gbias_ref[...]

    ones = ones_ref[...]
    s1 = jnp.dot(h.astype(bf16), ones, preferred_element_type=f32)
    h2 = h * h
    s2 = jnp.dot(h2.astype(bf16), ones, preferred_element_type=f32)
    mu = s1 * (1.0 / (_H * _C))
    var = s2 * (1.0 / (_H * _C)) - mu * mu
    out_ref[...] = (h - mu) * jax.lax.rsqrt(var + 1e-5) * lnw_ref[...] \
        + lnb_ref[...]

    attn_ref[0] = jnp.concatenate([a[e][0:1, :] for e in range(_E)], axis=1)


def _att_selector(att):
    eye = jnp.eye(_H, dtype=att.dtype)
    sel = eye[:, None, :] * att[:, :, None]
    return sel.reshape(_H * _C, _H).astype(jnp.bfloat16)


def kernel(striker_identity, striker_state, bowler_identity, bowler_state,
           partnership, h_state, Wc, bc, Wl, bl, Wr, br, att, gat_bias,
           ln_w, ln_b):
    B, D = h_state.shape
    HC = Wl.shape[0]
    grid = B // _TB
    row = lambda v: v.reshape(1, -1)

    h_actor, attn_all = pl.pallas_call(
        _gat_kernel,
        grid=(grid,),
        in_specs=[
            pl.BlockSpec((_TB, D), lambda i: (i, 0)),
            pl.BlockSpec((_TB, D), lambda i: (i, 0)),
            pl.BlockSpec((_TB, D), lambda i: (i, 0)),
            pl.BlockSpec((_TB, D), lambda i: (i, 0)),
            pl.BlockSpec((_TB, D), lambda i: (i, 0)),
            pl.BlockSpec((_TB, D), lambda i: (i, 0)),
            pl.BlockSpec((D, D), lambda i: (0, 0)),
            pl.BlockSpec((1, D), lambda i: (0, 0)),
            pl.BlockSpec((D, HC), lambda i: (0, 0)),
            pl.BlockSpec((1, HC), lambda i: (0, 0)),
            pl.BlockSpec((D, HC), lambda i: (0, 0)),
            pl.BlockSpec((1, HC), lambda i: (0, 0)),
            pl.BlockSpec((HC, _H), lambda i: (0, 0)),
            pl.BlockSpec((_H, HC), lambda i: (0, 0)),
            pl.BlockSpec((HC, 1), lambda i: (0, 0)),
            pl.BlockSpec((1, HC), lambda i: (0, 0)),
            pl.BlockSpec((1, D), lambda i: (0, 0)),
            pl.BlockSpec((1, D), lambda i: (0, 0)),
        ],
        out_specs=[
            pl.BlockSpec((_TB, D), lambda i: (i, 0)),
            pl.BlockSpec((1, 1, _E * _H), lambda i: (i, 0, 0)),
        ],
        out_shape=[
            jax.ShapeDtypeStruct((B, D), jnp.float32),
            jax.ShapeDtypeStruct((grid, 1, _E * _H), jnp.float32),
        ],
    )(striker_identity, striker_state, bowler_identity, bowler_state,
      partnership, h_state, Wc.T.astype(jnp.bfloat16), row(bc),
      Wl.T.astype(jnp.bfloat16), row(bl), Wr.T.astype(jnp.bfloat16), row(br),
      _att_selector(att),
      jnp.repeat(jnp.eye(_H, dtype=jnp.bfloat16), _C, axis=1),
      jnp.ones((HC, 1), dtype=jnp.bfloat16),
      row(gat_bias), row(ln_w), row(ln_b))

    attn0 = attn_all[0, 0].reshape(_E, _H)
    return h_actor, attn0


# speedup vs baseline: 4.4551x; 1.0587x over previous
import jax
import jax.numpy as jnp
from jax.experimental import pallas as pl

_EDGE_SRC = (0, 1, 2, 3, 0, 2, 1, 4, 3, 4)
_EDGE_DST = (1, 0, 3, 2, 2, 0, 4, 1, 4, 3)
_N = 5
_E = 10
_H = 4
_C = 64
_TB = 1024


def _gat_kernel(s_id_ref, s_st_ref, b_id_ref, b_st_ref, part_ref, hs_ref,
                WcT_ref, bc_ref, WlT_ref, bl_ref, WrT_ref, br_ref,
                att_ref, bcast_ref, ones_ref, gbias_ref, lnw_ref, lnb_ref,
                out_ref, attn_ref):
    f32 = jnp.float32
    bf16 = jnp.bfloat16
    hs = hs_ref[...]
    tb = hs.shape[0]
    ctx = jnp.dot(hs.astype(bf16), WcT_ref[...],
                  preferred_element_type=f32) + bc_ref[...]

    WlT = WlT_ref[...]
    WrT = WrT_ref[...]
    bl = bl_ref[...]
    br = br_ref[...]
    xl = []
    xr = []
    for ref in (s_id_ref, s_st_ref, b_id_ref, b_st_ref, part_ref):
        n = (ref[...] + ctx).astype(bf16)
        xl.append(jnp.dot(n, WlT, preferred_element_type=f32) + bl)
        xr.append(jnp.dot(n, WrT, preferred_element_type=f32) + br)

    A_att = att_ref[...]
    xl_bf = [x.astype(bf16) for x in xl]
    xr_bf = [x.astype(bf16) for x in xr]
    alphas = []
    for e in range(_E):
        pre = xl_bf[_EDGE_SRC[e]] + xr_bf[_EDGE_DST[e]]
        act = jnp.maximum(pre, 0.2 * pre)
        alphas.append(jnp.dot(act, A_att, preferred_element_type=f32))

    in_edges = [[] for _ in range(_N)]
    for e, d in enumerate(_EDGE_DST):
        in_edges[d].append(e)
    a = [None] * _E
    for d in range(_N):
        e1, e2 = in_edges[d]
        m = jnp.maximum(alphas[e1], alphas[e2])
        x1 = jnp.exp(alphas[e1] - m)
        x2 = jnp.exp(alphas[e2] - m)
        inv = 1.0 / (x1 + x2 + 1e-16)
        a[e1] = x1 * inv
        a[e2] = x2 * inv

    bcast = bcast_ref[...]
    acc = None
    for i in range(_N):
        es = [e for e in range(_E) if _EDGE_SRC[e] == i]
        w = a[es[0]] + a[es[1]]
        wb = jnp.dot(w.astype(bf16), bcast, preferred_element_type=f32)
        term = xl[i] * wb
        acc = term if acc is None else acc + term
    h = acc * 0.2 + gbias_ref[...]

    ones = ones_ref[...]
    s1 = jnp.dot(h.astype(bf16), ones, preferred_element_type=f32)
    h2 = h * h
    s2 = jnp.dot(h2.astype(bf16), ones, preferred_element_type=f32)
    mu = s1 * (1.0 / (_H * _C))
    var = s2 * (1.0 / (_H * _C)) - mu * mu
    out_ref[...] = (h - mu) * jax.lax.rsqrt(var + 1e-5) * lnw_ref[...] \
        + lnb_ref[...]

    attn_ref[0] = jnp.concatenate([a[e][0:1, :] for e in range(_E)], axis=1)


def _att_selector(att):
    eye = jnp.eye(_H, dtype=att.dtype)
    sel = eye[:, None, :] * att[:, :, None]
    return sel.reshape(_H * _C, _H).astype(jnp.bfloat16)


def kernel(striker_identity, striker_state, bowler_identity, bowler_state,
           partnership, h_state, Wc, bc, Wl, bl, Wr, br, att, gat_bias,
           ln_w, ln_b):
    B, D = h_state.shape
    HC = Wl.shape[0]
    grid = B // _TB
    row = lambda v: v.reshape(1, -1)

    h_actor, attn_all = pl.pallas_call(
        _gat_kernel,
        grid=(grid,),
        in_specs=[
            pl.BlockSpec((_TB, D), lambda i: (i, 0)),
            pl.BlockSpec((_TB, D), lambda i: (i, 0)),
            pl.BlockSpec((_TB, D), lambda i: (i, 0)),
            pl.BlockSpec((_TB, D), lambda i: (i, 0)),
            pl.BlockSpec((_TB, D), lambda i: (i, 0)),
            pl.BlockSpec((_TB, D), lambda i: (i, 0)),
            pl.BlockSpec((D, D), lambda i: (0, 0)),
            pl.BlockSpec((1, D), lambda i: (0, 0)),
            pl.BlockSpec((D, HC), lambda i: (0, 0)),
            pl.BlockSpec((1, HC), lambda i: (0, 0)),
            pl.BlockSpec((D, HC), lambda i: (0, 0)),
            pl.BlockSpec((1, HC), lambda i: (0, 0)),
            pl.BlockSpec((HC, _H), lambda i: (0, 0)),
            pl.BlockSpec((_H, HC), lambda i: (0, 0)),
            pl.BlockSpec((HC, 1), lambda i: (0, 0)),
            pl.BlockSpec((1, HC), lambda i: (0, 0)),
            pl.BlockSpec((1, D), lambda i: (0, 0)),
            pl.BlockSpec((1, D), lambda i: (0, 0)),
        ],
        out_specs=[
            pl.BlockSpec((_TB, D), lambda i: (i, 0)),
            pl.BlockSpec((1, 1, _E * _H), lambda i: (i, 0, 0)),
        ],
        out_shape=[
            jax.ShapeDtypeStruct((B, D), jnp.float32),
            jax.ShapeDtypeStruct((grid, 1, _E * _H), jnp.float32),
        ],
    )(striker_identity, striker_state, bowler_identity, bowler_state,
      partnership, h_state, Wc.T.astype(jnp.bfloat16), row(bc),
      Wl.T.astype(jnp.bfloat16), row(bl), Wr.T.astype(jnp.bfloat16), row(br),
      _att_selector(att),
      jnp.repeat(jnp.eye(_H, dtype=jnp.bfloat16), _C, axis=1),
      jnp.ones((HC, 1), dtype=jnp.bfloat16),
      row(gat_bias), row(ln_w), row(ln_b))

    attn0 = attn_all[0, 0].reshape(_E, _H)
    return h_actor, attn0
